# baseline (speedup 1.0000x reference)
import functools

import jax
import jax.numpy as jnp
from jax.experimental import pallas as pl
from jax.experimental.pallas import tpu as pltpu

_LANES = 8192
_RB = 8


def _loss_kernel(en_ref, de_ref, out_ref, acc_ref, *, nb_blocks):
    b = pl.program_id(0)
    j = pl.program_id(1)

    @pl.when(j == 0)
    def _init():
        acc_ref[...] = jnp.zeros_like(acc_ref)

    en = en_ref[0]
    de = de_ref[0]

    ed = (en * de).reshape(_RB, _LANES // 128, 128).sum(axis=1)
    ee = (en * en).reshape(_RB, _LANES // 128, 128).sum(axis=1)
    dd = (de * de).reshape(_RB, _LANES // 128, 128).sum(axis=1)
    acc_ref[0] += ed
    acc_ref[1] += ee
    acc_ref[2] += dd

    @pl.when(j == nb_blocks - 1)
    def _finalize():
        dot = jnp.sum(acc_ref[0])
        na2 = jnp.sum(acc_ref[1])
        nb2 = jnp.sum(acc_ref[2])
        term = 1.0 - dot / jnp.maximum(jnp.sqrt(na2) * jnp.sqrt(nb2), 1e-8)

        @pl.when(b == 0)
        def _first():
            out_ref[0, 0] = term

        @pl.when(b > 0)
        def _rest():
            out_ref[0, 0] += term


def kernel(encoder_features, decoder_features, global_step):
    B = encoder_features.shape[0]
    n = encoder_features.size // B
    rows = n // _LANES
    nb_blocks = rows // _RB

    en = encoder_features.reshape(B, rows, _LANES)
    de = decoder_features.reshape(B, rows, _LANES)

    out = pl.pallas_call(
        functools.partial(_loss_kernel, nb_blocks=nb_blocks),
        grid=(B, nb_blocks),
        in_specs=[
            pl.BlockSpec((1, _RB, _LANES), lambda b, j: (b, j, 0)),
            pl.BlockSpec((1, _RB, _LANES), lambda b, j: (b, j, 0)),
        ],
        out_specs=pl.BlockSpec(
            (1, 1), lambda b, j: (0, 0), memory_space=pltpu.SMEM
        ),
        out_shape=jax.ShapeDtypeStruct((1, 1), jnp.float32),
        scratch_shapes=[pltpu.VMEM((3, _RB, 128), jnp.float32)],
    )(en, de)
    return (out[0, 0] / B).reshape(())


# speedup vs baseline: 1.2082x; 1.2082x over previous
import jax
import jax.numpy as jnp
from jax.experimental import pallas as pl
from jax.experimental.pallas import tpu as pltpu

_LANES = 8192


def _loss_kernel(en_ref, de_ref, out_ref):
    b = pl.program_id(0)
    en = en_ref[0]
    de = de_ref[0]

    dot = jnp.sum(en * de)
    na2 = jnp.sum(en * en)
    nb2 = jnp.sum(de * de)
    term = 1.0 - dot / jnp.maximum(jnp.sqrt(na2) * jnp.sqrt(nb2), 1e-8)

    @pl.when(b == 0)
    def _first():
        out_ref[0, 0] = term

    @pl.when(b > 0)
    def _rest():
        out_ref[0, 0] += term


def kernel(encoder_features, decoder_features, global_step):
    B = encoder_features.shape[0]
    n = encoder_features.size // B
    rows = n // _LANES

    en = encoder_features.reshape(B, rows, _LANES)
    de = decoder_features.reshape(B, rows, _LANES)

    out = pl.pallas_call(
        _loss_kernel,
        grid=(B,),
        in_specs=[
            pl.BlockSpec((1, rows, _LANES), lambda b: (b, 0, 0)),
            pl.BlockSpec((1, rows, _LANES), lambda b: (b, 0, 0)),
        ],
        out_specs=pl.BlockSpec(
            (1, 1), lambda b: (0, 0), memory_space=pltpu.SMEM
        ),
        out_shape=jax.ShapeDtypeStruct((1, 1), jnp.float32),
    )(en, de)
    return (out[0, 0] / B).reshape(())


# speedup vs baseline: 1.2637x; 1.0460x over previous
import jax
import jax.numpy as jnp
from jax.experimental import pallas as pl
from jax.experimental.pallas import tpu as pltpu

_LANES = 128


def _loss_kernel(en_ref, de_ref, out_ref):
    b = pl.program_id(0)
    en = en_ref[0]
    de = de_ref[0]

    dot = jnp.sum(en * de)
    na2 = jnp.sum(en * en)
    nb2 = jnp.sum(de * de)
    term = 1.0 - dot / jnp.maximum(jnp.sqrt(na2) * jnp.sqrt(nb2), 1e-8)

    @pl.when(b == 0)
    def _first():
        out_ref[0, 0] = term

    @pl.when(b > 0)
    def _rest():
        out_ref[0, 0] += term


def kernel(encoder_features, decoder_features, global_step):
    B = encoder_features.shape[0]
    n = encoder_features.size // B
    rows = n // _LANES

    en = encoder_features.reshape(B, rows, _LANES)
    de = decoder_features.reshape(B, rows, _LANES)

    out = pl.pallas_call(
        _loss_kernel,
        grid=(B,),
        in_specs=[
            pl.BlockSpec((1, rows, _LANES), lambda b: (b, 0, 0)),
            pl.BlockSpec((1, rows, _LANES), lambda b: (b, 0, 0)),
        ],
        out_specs=pl.BlockSpec(
            (1, 1), lambda b: (0, 0), memory_space=pltpu.SMEM
        ),
        out_shape=jax.ShapeDtypeStruct((1, 1), jnp.float32),
    )(en, de)
    return (out[0, 0] / B).reshape(())
